# baseline (speedup 1.0000x reference)
import functools

import jax
import jax.numpy as jnp
from jax import lax
from jax.experimental import pallas as pl
from jax.experimental.pallas import tpu as pltpu
from jax.experimental.pallas import tpu_sc as plsc


AX0 = 64
AX1 = 64
DIM = 1024
SEQ = AX0 * AX1
BATCH = 4
LANES = 16
NC = 2
NS = 16
NW = NC * NS

SC_BATCH = 2
TC_BATCH = BATCH - SC_BATCH

I_PER_W = AX0 // (NW // 2)
J_HALF = AX1 // 2


def _sc_body(w0_hbm, w1_hbm, out_hbm, w1_v, w0_v, buf0, buf1, sem0, sem1):
    wid = lax.axis_index("s") * NC + lax.axis_index("c")
    h = wid % 2
    g = wid // 2
    i_base = g * I_PER_W
    row_off = h * J_HALF

    pltpu.sync_copy(w1_hbm.at[pl.ds(row_off, J_HALF)], w1_v)
    pltpu.sync_copy(w0_hbm.at[pl.ds(i_base, I_PER_W)], w0_v)

    pending = {0: [], 1: []}
    for k in range(I_PER_W):
        slot = k % 2
        buf = buf0 if slot == 0 else buf1
        sem = sem0 if slot == 0 else sem1
        for cp in pending[slot]:
            cp.wait()
        pending[slot] = []

        @plsc.parallel_loop(0, J_HALF)
        def _(j, k=k, buf=buf):
            for d in range(DIM // LANES):
                sl = pl.ds(d * LANES, LANES)
                buf[j, sl] = w0_v[k, sl] + w1_v[j, sl]

        for b in range(SC_BATCH):
            row = b * SEQ + (i_base + k) * AX1 + row_off
            cp = pltpu.make_async_copy(
                buf, out_hbm.at[pl.ds(row, J_HALF)], sem
            )
            cp.start()
            pending[slot].append(cp)

    for slot in (0, 1):
        for cp in pending[slot]:
            cp.wait()


@functools.partial(
    pl.kernel,
    mesh=plsc.VectorSubcoreMesh(core_axis_name="c", subcore_axis_name="s"),
    out_type=jax.ShapeDtypeStruct((SC_BATCH * SEQ, DIM), jnp.float32),
    scratch_types=[
        pltpu.VMEM((J_HALF, DIM), jnp.float32),
        pltpu.VMEM((I_PER_W, DIM), jnp.float32),
        pltpu.VMEM((J_HALF, DIM), jnp.float32),
        pltpu.VMEM((J_HALF, DIM), jnp.float32),
        pltpu.SemaphoreType.DMA,
        pltpu.SemaphoreType.DMA,
    ],
)
def _sc_kernel(w0_hbm, w1_hbm, out_hbm, w1_v, w0_v, buf0, buf1, sem0, sem1):
    _sc_body(w0_hbm, w1_hbm, out_hbm, w1_v, w0_v, buf0, buf1, sem0, sem1)


TC_I_BLK = 8


def _tc_body(w0_ref, w1_ref, o_ref):
    w0b = w0_ref[...]
    w1b = w1_ref[...]
    o_ref[...] = (w0b[:, None, :] + w1b[None, :, :]).reshape(
        1, TC_I_BLK * AX1, DIM
    )


def _tc_kernel(w0f, w1f):
    return pl.pallas_call(
        _tc_body,
        grid=(TC_BATCH, AX0 // TC_I_BLK),
        in_specs=[
            pl.BlockSpec((TC_I_BLK, DIM), lambda b, i: (i, 0)),
            pl.BlockSpec((AX1, DIM), lambda b, i: (0, 0)),
        ],
        out_specs=pl.BlockSpec(
            (1, TC_I_BLK * AX1, DIM), lambda b, i: (b, i, 0)
        ),
        out_shape=jax.ShapeDtypeStruct((TC_BATCH, SEQ, DIM), jnp.float32),
    )(w0f, w1f)


def kernel(x, w0, w1):
    w0f = w0.reshape(AX0, DIM)
    w1f = w1.reshape(AX1, DIM)
    sc_part = _sc_kernel(w0f, w1f).reshape(SC_BATCH, SEQ, DIM)
    tc_part = _tc_kernel(w0f, w1f)
    out = jnp.concatenate([tc_part, sc_part], axis=0)
    return out.astype(x.dtype)


# speedup vs baseline: 3.8010x; 3.8010x over previous
import jax
import jax.numpy as jnp
from jax.experimental import pallas as pl


AX0 = 64
AX1 = 64
DIM = 1024
SEQ = AX0 * AX1
BATCH = 4
I_BLK = 16


def _body(w0_ref, w1_ref, o_ref):
    w0b = w0_ref[...]
    w1b = w1_ref[...]
    o_ref[...] = (w0b[:, None, :] + w1b[None, :, :]).reshape(
        1, I_BLK * AX1, DIM
    )


def kernel(x, w0, w1):
    w0f = w0.reshape(AX0, DIM)
    w1f = w1.reshape(AX1, DIM)
    out = pl.pallas_call(
        _body,
        grid=(BATCH, AX0 // I_BLK),
        in_specs=[
            pl.BlockSpec((I_BLK, DIM), lambda b, i: (i, 0)),
            pl.BlockSpec((AX1, DIM), lambda b, i: (0, 0)),
        ],
        out_specs=pl.BlockSpec((1, I_BLK * AX1, DIM), lambda b, i: (b, i, 0)),
        out_shape=jax.ShapeDtypeStruct((BATCH, SEQ, DIM), x.dtype),
    )(w0f, w1f)
    return out
